# baseline (speedup 1.0000x reference)
import functools

import jax
import jax.numpy as jnp
from jax import lax
from jax.experimental import pallas as pl
from jax.experimental.pallas import tpu as pltpu, tpu_sc as plsc

N = 10000
D = 128
D2 = D // 2
E = 320000
NLAYERS = 5
G = 64
C = 10
EPS = 1e-5

NS = 16
EPT = E // NS
K = 125
CHUNKS = EPT // K
NBUF = 8
LOOK = NBUF - 2
IB = 40
NBLK = CHUNKS // IB
ACC_N = 10240
RPT = ACC_N // NS
RSTG = 80


def _agg_body(hs_hbm, src_hbm, dst_hbm, out_hbm,
              src_v, dst_v, rows_v, acc_sh,
              sem_g, sem_i, *sems):
    cid = lax.axis_index("c")
    sid = lax.axis_index("s")
    base = sid * RPT
    h_half = hs_hbm.at[cid]

    def zrow(r, carry):
        for c in range(D2 // 16):
            rows_v[0, r, pl.ds(c * 16, 16)] = jnp.zeros((16,), jnp.float32)
        return carry
    lax.fori_loop(0, RSTG, zrow, 0)
    stage = rows_v.at[0, pl.ds(0, RSTG)]
    for t in range(RPT // RSTG):
        pltpu.sync_copy(stage, acc_sh.at[pl.ds(base + t * RSTG, RSTG)])
    plsc.subcore_barrier()

    def idx_start(g, ib):
        pltpu.make_async_copy(src_hbm.at[sid * NBLK + g], src_v.at[ib],
                              sem_i).start()
        pltpu.make_async_copy(dst_hbm.at[sid * NBLK + g], dst_v.at[ib],
                              sem_i).start()

    def idx_wait(ib):
        pltpu.make_async_copy(src_hbm.at[0], src_v.at[ib], sem_i).wait()
        pltpu.make_async_copy(dst_hbm.at[0], dst_v.at[ib], sem_i).wait()

    def gather_start(ib, j, b):
        pltpu.make_async_copy(h_half.at[src_v.at[ib, j]], rows_v.at[b],
                              sem_g).start()

    def gather_wait(b):
        pltpu.make_async_copy(h_half.at[src_v.at[0, 0]], rows_v.at[b],
                              sem_g).wait()

    def scatter_start(ib, j, b):
        pltpu.make_async_copy(rows_v.at[b], acc_sh.at[dst_v.at[ib, j]],
                              sems[b]).start(add=True)

    def scatter_wait(b):
        pltpu.make_async_copy(h_half.at[src_v.at[0, 0]], rows_v.at[b],
                              sems[b]).wait()

    idx_start(0, 0)
    idx_wait(0)

    for g in range(NBLK):
        ib = g % 2
        if g + 1 < NBLK:
            idx_start(g + 1, (g + 1) % 2)

        for j in range(LOOK):
            gather_start(ib, j, j % NBUF)

        for b in range(NBUF):
            gather_wait(b)
            scatter_start(ib, b, b)
            if b >= 2:
                scatter_wait((b + LOOK) % NBUF)
            gather_start(ib, b + LOOK, (b + LOOK) % NBUF)

        def round_(j0, carry):
            for b in range(NBUF):
                j = j0 * NBUF + b
                gather_wait(b)
                scatter_start(ib, j, b)
                b2 = (b + LOOK) % NBUF
                scatter_wait(b2)
                gather_start(ib, j + LOOK, b2)
            return carry
        lax.fori_loop(1, IB // NBUF - 1, round_, 0)

        if g + 1 < NBLK:
            idx_wait((g + 1) % 2)

        for b in range(NBUF):
            j = IB - NBUF + b
            gather_wait(b)
            scatter_start(ib, j, b)
            scatter_wait((b + LOOK) % NBUF)
            if b < NBUF - LOOK:
                gather_start(ib, j + LOOK, (b + LOOK) % NBUF)

        for b in range(NBUF - LOOK):
            scatter_wait((IB - NBUF + LOOK + b) % NBUF)

    plsc.subcore_barrier()

    for t in range(RPT // RSTG):
        pltpu.sync_copy(acc_sh.at[pl.ds(base + t * RSTG, RSTG)], stage)
        pltpu.sync_copy(
            stage, out_hbm.at[cid].at[pl.ds(base + t * RSTG, RSTG)])


_agg = functools.partial(
    pl.kernel,
    out_type=jax.ShapeDtypeStruct((2, ACC_N, D2), jnp.float32),
    mesh=plsc.VectorSubcoreMesh(core_axis_name="c", subcore_axis_name="s"),
    compiler_params=pltpu.CompilerParams(use_tc_tiling_on_sc=False),
    scratch_types=[
        pltpu.VMEM((2, IB, K), jnp.int32),
        pltpu.VMEM((2, IB, K), jnp.int32),
        pltpu.VMEM((NBUF, K, D2), jnp.float32),
        pltpu.VMEM_SHARED((ACC_N, D2), jnp.float32),
    ] + [pltpu.SemaphoreType.DMA] * (2 + NBUF),
)(_agg_body)


NB = 10
R = N // NB


def _mlp_body(hs_ref, p0_ref, p1_ref, w1, b1, g1, bt1, w2, b2, g2, bt2,
              o_ref, y_sc, s1, s2, s3, s4):
    p = pl.program_id(0)
    i = pl.program_id(1)
    rows = pl.ds(i * R, R)

    @pl.when((p == 0) & (i == 0))
    def _():
        s1[...] = jnp.zeros_like(s1)
        s2[...] = jnp.zeros_like(s2)
        s3[...] = jnp.zeros_like(s3)
        s4[...] = jnp.zeros_like(s4)

    @pl.when(p == 0)
    def _():
        m = jnp.concatenate([hs_ref[0] + p0_ref[0], hs_ref[1] + p1_ref[0]],
                            axis=1)
        y = jnp.dot(m, w1[...], preferred_element_type=jnp.float32) + b1[...]
        s1[...] += jnp.sum(y, axis=0, keepdims=True)
        s2[...] += jnp.sum(y * y, axis=0, keepdims=True)
        y_sc[rows, :] = y

    @pl.when((p == 1) & (i == 0))
    def _():
        mu = s1[...] / N
        var = s2[...] / N - mu * mu
        scale = g1[...] * lax.rsqrt(var + EPS)
        s1[...] = scale
        s2[...] = bt1[...] - mu * scale

    @pl.when(p == 1)
    def _():
        y = jnp.maximum(y_sc[rows, :] * s1[...] + s2[...], 0.0)
        z = jnp.dot(y, w2[...], preferred_element_type=jnp.float32) + b2[...]
        z = jnp.maximum(z, 0.0)
        s3[...] += jnp.sum(z, axis=0, keepdims=True)
        s4[...] += jnp.sum(z * z, axis=0, keepdims=True)
        y_sc[rows, :] = z

    @pl.when((p == 2) & (i == 0))
    def _():
        mu = s3[...] / N
        var = s4[...] / N - mu * mu
        scale = g2[...] * lax.rsqrt(var + EPS)
        s3[...] = scale
        s4[...] = bt2[...] - mu * scale

    @pl.when(p == 2)
    def _():
        z = jnp.maximum(y_sc[rows, :] * s3[...] + s4[...], 0.0)
        o_ref[0] = lax.slice(z, (0, 0), (R, D2))
        o_ref[1] = lax.slice(z, (0, D2), (R, D))


def _first_phase_map(p, i):
    return (0, jnp.where(p == 0, i, NB - 1), 0)


_mlp = pl.pallas_call(
    _mlp_body,
    grid=(3, NB),
    in_specs=[
        pl.BlockSpec((2, R, D2), _first_phase_map),
        pl.BlockSpec((1, R, D2), _first_phase_map),
        pl.BlockSpec((1, R, D2),
                     lambda p, i: (1, jnp.where(p == 0, i, NB - 1), 0)),
        pl.BlockSpec((D, D), lambda p, i: (0, 0)),
        pl.BlockSpec((1, D), lambda p, i: (0, 0)),
        pl.BlockSpec((1, D), lambda p, i: (0, 0)),
        pl.BlockSpec((1, D), lambda p, i: (0, 0)),
        pl.BlockSpec((D, D), lambda p, i: (0, 0)),
        pl.BlockSpec((1, D), lambda p, i: (0, 0)),
        pl.BlockSpec((1, D), lambda p, i: (0, 0)),
        pl.BlockSpec((1, D), lambda p, i: (0, 0)),
    ],
    out_specs=pl.BlockSpec((2, R, D2), lambda p, i: (0, i, 0)),
    out_shape=jax.ShapeDtypeStruct((2, N, D2), jnp.float32),
    scratch_shapes=[
        pltpu.VMEM((N, D), jnp.float32),
        pltpu.VMEM((1, D), jnp.float32),
        pltpu.VMEM((1, D), jnp.float32),
        pltpu.VMEM((1, D), jnp.float32),
        pltpu.VMEM((1, D), jnp.float32),
    ],
)


def _pool_body(hs_ref, batch_ref, fcw_ref, fcb_ref, o_ref):
    h = jnp.concatenate([hs_ref[0], hs_ref[1]], axis=1)
    gids = lax.broadcasted_iota(jnp.int32, (N, 128), 1)
    onehot = (batch_ref[...] == gids).astype(jnp.float32)
    pooled = lax.dot_general(onehot, h, (((0,), (0,)), ((), ())),
                             preferred_element_type=jnp.float32)
    out = jnp.dot(pooled, fcw_ref[...],
                  preferred_element_type=jnp.float32) + fcb_ref[...]
    o_ref[...] = out[:G, :]


_pool = pl.pallas_call(
    _pool_body,
    out_shape=jax.ShapeDtypeStruct((G, C), jnp.float32),
)


def kernel(x, edge_index, batch, W1, b1, g1, bt1, W2, b2, g2, bt2, fcW, fcb):
    src = edge_index[0].reshape(NS * NBLK, IB, K)
    dst = edge_index[1].reshape(NS * NBLK, IB, K)
    batch2d = batch.reshape(N, 1)
    hs = jnp.stack([x[:, :D2], x[:, D2:]])
    for i in range(NLAYERS):
        parts = _agg(hs, src, dst)
        hs = _mlp(hs, parts, parts,
                  W1[i], b1[i].reshape(1, D), g1[i].reshape(1, D),
                  bt1[i].reshape(1, D),
                  W2[i], b2[i].reshape(1, D), g2[i].reshape(1, D),
                  bt2[i].reshape(1, D))
    return _pool(hs, batch2d, fcW, fcb.reshape(1, C))


# speedup vs baseline: 1.0287x; 1.0287x over previous
import functools

import jax
import jax.numpy as jnp
from jax import lax
from jax.experimental import pallas as pl
from jax.experimental.pallas import tpu as pltpu, tpu_sc as plsc

N = 10000
D = 128
D2 = D // 2
E = 320000
NLAYERS = 5
G = 64
C = 10
EPS = 1e-5

NS = 16
EPT = E // NS
K = 125
CHUNKS = EPT // K
NBUF = 8
LOOK = NBUF - 2
IB = 40
NBLK = CHUNKS // IB
ACC_N = 10240
RPT = ACC_N // NS
RSTG = 80


def _agg_body(hs_hbm, src_hbm, dst_hbm, out_hbm,
              src_v, dst_v, rows_v, acc_sh,
              sem_g, sem_i, *sems):
    cid = lax.axis_index("c")
    sid = lax.axis_index("s")
    base = sid * RPT
    h_half = hs_hbm.at[cid]

    def zrow(r, carry):
        for c in range(D2 // 16):
            rows_v[0, r, pl.ds(c * 16, 16)] = jnp.zeros((16,), jnp.float32)
        return carry
    lax.fori_loop(0, RSTG, zrow, 0)
    stage = rows_v.at[0, pl.ds(0, RSTG)]
    for t in range(RPT // RSTG):
        pltpu.sync_copy(stage, acc_sh.at[pl.ds(base + t * RSTG, RSTG)])
    plsc.subcore_barrier()

    def idx_start(g, ib):
        pltpu.make_async_copy(src_hbm.at[sid * NBLK + g], src_v.at[ib],
                              sem_i).start()
        pltpu.make_async_copy(dst_hbm.at[sid * NBLK + g], dst_v.at[ib],
                              sem_i).start()

    def idx_wait(ib):
        pltpu.make_async_copy(src_hbm.at[0], src_v.at[ib], sem_i).wait()
        pltpu.make_async_copy(dst_hbm.at[0], dst_v.at[ib], sem_i).wait()

    def gather_start(ib, j, b):
        pltpu.make_async_copy(h_half.at[src_v.at[ib, j]], rows_v.at[b],
                              sem_g).start()

    def gather_wait(b):
        pltpu.make_async_copy(h_half.at[src_v.at[0, 0]], rows_v.at[b],
                              sem_g).wait()

    def scatter_start(ib, j, b):
        pltpu.make_async_copy(rows_v.at[b], acc_sh.at[dst_v.at[ib, j]],
                              sems[b]).start(add=True)

    def scatter_wait(b):
        pltpu.make_async_copy(h_half.at[src_v.at[0, 0]], rows_v.at[b],
                              sems[b]).wait()

    idx_start(0, 0)
    idx_wait(0)

    for g in range(NBLK):
        ib = g % 2
        if g + 1 < NBLK:
            idx_start(g + 1, (g + 1) % 2)

        for j in range(LOOK):
            gather_start(ib, j, j % NBUF)

        for b in range(NBUF):
            gather_wait(b)
            scatter_start(ib, b, b)
            if b >= 2:
                scatter_wait((b + LOOK) % NBUF)
            gather_start(ib, b + LOOK, (b + LOOK) % NBUF)

        def round_(j0, carry):
            for b in range(NBUF):
                j = j0 * NBUF + b
                gather_wait(b)
                scatter_start(ib, j, b)
                b2 = (b + LOOK) % NBUF
                scatter_wait(b2)
                gather_start(ib, j + LOOK, b2)
            return carry
        lax.fori_loop(1, IB // NBUF - 1, round_, 0)

        if g + 1 < NBLK:
            idx_wait((g + 1) % 2)

        for b in range(NBUF):
            j = IB - NBUF + b
            gather_wait(b)
            scatter_start(ib, j, b)
            scatter_wait((b + LOOK) % NBUF)
            if b < NBUF - LOOK:
                gather_start(ib, j + LOOK, (b + LOOK) % NBUF)

        for b in range(NBUF - LOOK):
            scatter_wait((IB - NBUF + LOOK + b) % NBUF)

    plsc.subcore_barrier()

    for t in range(RPT // RSTG):
        pltpu.sync_copy(acc_sh.at[pl.ds(base + t * RSTG, RSTG)], stage)
        pltpu.sync_copy(
            stage, out_hbm.at[cid].at[pl.ds(base + t * RSTG, RSTG)])


_agg = functools.partial(
    pl.kernel,
    out_type=jax.ShapeDtypeStruct((2, ACC_N, D2), jnp.float32),
    mesh=plsc.VectorSubcoreMesh(core_axis_name="c", subcore_axis_name="s"),
    compiler_params=pltpu.CompilerParams(use_tc_tiling_on_sc=False),
    scratch_types=[
        pltpu.VMEM((2, IB, K), jnp.int32),
        pltpu.VMEM((2, IB, K), jnp.int32),
        pltpu.VMEM((NBUF, K, D2), jnp.float32),
        pltpu.VMEM_SHARED((ACC_N, D2), jnp.float32),
    ] + [pltpu.SemaphoreType.DMA] * (2 + NBUF),
)(_agg_body)


NB = 10
R = N // NB


def _mlp_body(hs_ref, p0_ref, p1_ref, w1, b1, g1, bt1, w2, b2, g2, bt2,
              o_ref, y_sc, s1, s2, s3, s4):
    p = pl.program_id(0)
    i = pl.program_id(1)
    rows = pl.ds(i * R, R)

    @pl.when((p == 0) & (i == 0))
    def _():
        s1[...] = jnp.zeros_like(s1)
        s2[...] = jnp.zeros_like(s2)
        s3[...] = jnp.zeros_like(s3)
        s4[...] = jnp.zeros_like(s4)

    @pl.when(p == 0)
    def _():
        m = jnp.concatenate([hs_ref[0] + p0_ref[0], hs_ref[1] + p1_ref[0]],
                            axis=1)
        y = jnp.dot(m, w1[...], preferred_element_type=jnp.float32) + b1[...]
        s1[...] += jnp.sum(y, axis=0, keepdims=True)
        s2[...] += jnp.sum(y * y, axis=0, keepdims=True)
        y_sc[rows, :] = y

    @pl.when((p == 1) & (i == 0))
    def _():
        mu = s1[...] / N
        var = s2[...] / N - mu * mu
        scale = g1[...] * lax.rsqrt(var + EPS)
        s1[...] = scale
        s2[...] = bt1[...] - mu * scale

    @pl.when(p == 1)
    def _():
        y = jnp.maximum(y_sc[rows, :] * s1[...] + s2[...], 0.0)
        z = jnp.dot(y, w2[...], preferred_element_type=jnp.float32) + b2[...]
        z = jnp.maximum(z, 0.0)
        s3[...] += jnp.sum(z, axis=0, keepdims=True)
        s4[...] += jnp.sum(z * z, axis=0, keepdims=True)
        y_sc[rows, :] = z

    @pl.when((p == 2) & (i == 0))
    def _():
        mu = s3[...] / N
        var = s4[...] / N - mu * mu
        scale = g2[...] * lax.rsqrt(var + EPS)
        s3[...] = scale
        s4[...] = bt2[...] - mu * scale

    @pl.when(p == 2)
    def _():
        z = jnp.maximum(y_sc[rows, :] * s3[...] + s4[...], 0.0)
        o_ref[0] = lax.slice(z, (0, 0), (R, D2))
        o_ref[1] = lax.slice(z, (0, D2), (R, D))


def _first_phase_map(p, i):
    return (0, jnp.where(p == 0, i, NB - 1), 0)


_mlp = pl.pallas_call(
    _mlp_body,
    grid=(3, NB),
    in_specs=[
        pl.BlockSpec((2, R, D2), _first_phase_map),
        pl.BlockSpec((1, R, D2), _first_phase_map),
        pl.BlockSpec((1, R, D2),
                     lambda p, i: (1, jnp.where(p == 0, i, NB - 1), 0)),
        pl.BlockSpec((D, D), lambda p, i: (0, 0)),
        pl.BlockSpec((1, D), lambda p, i: (0, 0)),
        pl.BlockSpec((1, D), lambda p, i: (0, 0)),
        pl.BlockSpec((1, D), lambda p, i: (0, 0)),
        pl.BlockSpec((D, D), lambda p, i: (0, 0)),
        pl.BlockSpec((1, D), lambda p, i: (0, 0)),
        pl.BlockSpec((1, D), lambda p, i: (0, 0)),
        pl.BlockSpec((1, D), lambda p, i: (0, 0)),
    ],
    out_specs=pl.BlockSpec((2, R, D2),
                           lambda p, i: (0, jnp.where(p == 2, i, 0), 0)),
    out_shape=jax.ShapeDtypeStruct((2, N, D2), jnp.float32),
    scratch_shapes=[
        pltpu.VMEM((N, D), jnp.float32),
        pltpu.VMEM((1, D), jnp.float32),
        pltpu.VMEM((1, D), jnp.float32),
        pltpu.VMEM((1, D), jnp.float32),
        pltpu.VMEM((1, D), jnp.float32),
    ],
)


def _pool_body(hs_ref, batch_ref, fcw_ref, fcb_ref, o_ref):
    h = jnp.concatenate([hs_ref[0], hs_ref[1]], axis=1)
    gids = lax.broadcasted_iota(jnp.int32, (N, 128), 1)
    onehot = (batch_ref[...] == gids).astype(jnp.float32)
    pooled = lax.dot_general(onehot, h, (((0,), (0,)), ((), ())),
                             preferred_element_type=jnp.float32)
    out = jnp.dot(pooled, fcw_ref[...],
                  preferred_element_type=jnp.float32) + fcb_ref[...]
    o_ref[...] = out[:G, :]


_pool = pl.pallas_call(
    _pool_body,
    out_shape=jax.ShapeDtypeStruct((G, C), jnp.float32),
)


def kernel(x, edge_index, batch, W1, b1, g1, bt1, W2, b2, g2, bt2, fcW, fcb):
    src = edge_index[0].reshape(NS * NBLK, IB, K)
    dst = edge_index[1].reshape(NS * NBLK, IB, K)
    batch2d = batch.reshape(N, 1)
    hs = jnp.stack([x[:, :D2], x[:, D2:]])
    for i in range(NLAYERS):
        parts = _agg(hs, src, dst)
        hs = _mlp(hs, parts, parts,
                  W1[i], b1[i].reshape(1, D), g1[i].reshape(1, D),
                  bt1[i].reshape(1, D),
                  W2[i], b2[i].reshape(1, D), g2[i].reshape(1, D),
                  bt2[i].reshape(1, D))
    return _pool(hs, batch2d, fcW, fcb.reshape(1, C))


# speedup vs baseline: 1.0648x; 1.0351x over previous
import functools

import jax
import jax.numpy as jnp
from jax import lax
from jax.experimental import pallas as pl
from jax.experimental.pallas import tpu as pltpu, tpu_sc as plsc

N = 10000
D = 128
D2 = D // 2
E = 320000
NLAYERS = 5
G = 64
C = 10
EPS = 1e-5

NS = 16
EPT = E // NS
K = 125
CHUNKS = EPT // K
NBUF = 8
LOOK = NBUF - 2
IB = 40
NBLK = CHUNKS // IB
ACC_N = 10240
RPT = ACC_N // NS
RSTG = 80


def _agg_body(hs_hbm, src_hbm, dst_hbm, out_hbm,
              src_v, dst_v, rows_v, acc_sh,
              sem_g, sem_i, *sems):
    cid = lax.axis_index("c")
    sid = lax.axis_index("s")
    base = sid * RPT
    h_half = hs_hbm.at[cid]

    def zrow(r, carry):
        for c in range(D2 // 16):
            rows_v[0, r, pl.ds(c * 16, 16)] = jnp.zeros((16,), jnp.float32)
        return carry
    lax.fori_loop(0, RSTG, zrow, 0)
    stage = rows_v.at[0, pl.ds(0, RSTG)]
    for t in range(RPT // RSTG):
        pltpu.sync_copy(stage, acc_sh.at[pl.ds(base + t * RSTG, RSTG)])
    plsc.subcore_barrier()

    def idx_start(g, ib):
        pltpu.make_async_copy(src_hbm.at[sid * NBLK + g], src_v.at[ib],
                              sem_i).start()
        pltpu.make_async_copy(dst_hbm.at[sid * NBLK + g], dst_v.at[ib],
                              sem_i).start()

    def idx_wait(ib):
        pltpu.make_async_copy(src_hbm.at[0], src_v.at[ib], sem_i).wait()
        pltpu.make_async_copy(dst_hbm.at[0], dst_v.at[ib], sem_i).wait()

    def gather_start(ib, j, b):
        pltpu.make_async_copy(h_half.at[src_v.at[ib, j]], rows_v.at[b],
                              sem_g).start()

    def gather_wait(b):
        pltpu.make_async_copy(h_half.at[src_v.at[0, 0]], rows_v.at[b],
                              sem_g).wait()

    def scatter_start(ib, j, b):
        pltpu.make_async_copy(rows_v.at[b], acc_sh.at[dst_v.at[ib, j]],
                              sems[b]).start(add=True)

    def scatter_wait(b):
        pltpu.make_async_copy(h_half.at[src_v.at[0, 0]], rows_v.at[b],
                              sems[b]).wait()

    idx_start(0, 0)
    idx_wait(0)

    for g in range(NBLK):
        ib = g % 2
        if g + 1 < NBLK:
            idx_start(g + 1, (g + 1) % 2)

        for j in range(LOOK):
            gather_start(ib, j, j % NBUF)

        for b in range(NBUF):
            gather_wait(b)
            scatter_start(ib, b, b)
            if b >= 2:
                scatter_wait((b + LOOK) % NBUF)
            gather_start(ib, b + LOOK, (b + LOOK) % NBUF)

        def round_(j0, carry):
            for b in range(NBUF):
                j = j0 * NBUF + b
                gather_wait(b)
                scatter_start(ib, j, b)
                b2 = (b + LOOK) % NBUF
                scatter_wait(b2)
                gather_start(ib, j + LOOK, b2)
            return carry
        lax.fori_loop(1, IB // NBUF - 1, round_, 0)

        if g + 1 < NBLK:
            idx_wait((g + 1) % 2)

        for b in range(NBUF):
            j = IB - NBUF + b
            gather_wait(b)
            scatter_start(ib, j, b)
            scatter_wait((b + LOOK) % NBUF)
            if b < NBUF - LOOK:
                gather_start(ib, j + LOOK, (b + LOOK) % NBUF)

        for b in range(NBUF - LOOK):
            scatter_wait((IB - NBUF + LOOK + b) % NBUF)

    plsc.subcore_barrier()

    for t in range(RPT // RSTG):
        pltpu.sync_copy(acc_sh.at[pl.ds(base + t * RSTG, RSTG)], stage)
        pltpu.sync_copy(
            stage, out_hbm.at[cid].at[pl.ds(base + t * RSTG, RSTG)])


_agg = functools.partial(
    pl.kernel,
    out_type=jax.ShapeDtypeStruct((2, ACC_N, D2), jnp.float32),
    mesh=plsc.VectorSubcoreMesh(core_axis_name="c", subcore_axis_name="s"),
    compiler_params=pltpu.CompilerParams(use_tc_tiling_on_sc=False),
    scratch_types=[
        pltpu.VMEM((2, IB, K), jnp.int32),
        pltpu.VMEM((2, IB, K), jnp.int32),
        pltpu.VMEM((NBUF, K, D2), jnp.float32),
        pltpu.VMEM_SHARED((ACC_N, D2), jnp.float32),
    ] + [pltpu.SemaphoreType.DMA] * (2 + NBUF),
)(_agg_body)


def _mlp_core(hs_ref, p_ref, w1, b1, g1, bt1, w2, b2, g2, bt2):
    h = jnp.concatenate([hs_ref[0], hs_ref[1]], axis=1)
    agg = jnp.concatenate(
        [lax.slice(p_ref[0], (0, 0), (N, D2)),
         lax.slice(p_ref[1], (0, 0), (N, D2))], axis=1)
    m = h + agg
    y = jnp.dot(m, w1[...], preferred_element_type=jnp.float32) + b1[...]
    mu = jnp.mean(y, axis=0, keepdims=True)
    var = jnp.mean((y - mu) * (y - mu), axis=0, keepdims=True)
    y = g1[...] * (y - mu) * lax.rsqrt(var + EPS) + bt1[...]
    y = jnp.maximum(y, 0.0)
    z = jnp.dot(y, w2[...], preferred_element_type=jnp.float32) + b2[...]
    z = jnp.maximum(z, 0.0)
    mu2 = jnp.mean(z, axis=0, keepdims=True)
    var2 = jnp.mean((z - mu2) * (z - mu2), axis=0, keepdims=True)
    z = g2[...] * (z - mu2) * lax.rsqrt(var2 + EPS) + bt2[...]
    return jnp.maximum(z, 0.0)


def _mlp_body(hs_ref, p_ref, w1, b1, g1, bt1, w2, b2, g2, bt2, o_ref):
    z = _mlp_core(hs_ref, p_ref, w1, b1, g1, bt1, w2, b2, g2, bt2)
    o_ref[0] = lax.slice(z, (0, 0), (N, D2))
    o_ref[1] = lax.slice(z, (0, D2), (N, D))


_mlp = pl.pallas_call(
    _mlp_body,
    out_shape=jax.ShapeDtypeStruct((2, N, D2), jnp.float32),
)


def _mlp_pool_body(hs_ref, p_ref, w1, b1, g1, bt1, w2, b2, g2, bt2,
                   batch_ref, fcw_ref, fcb_ref, o_ref):
    z = _mlp_core(hs_ref, p_ref, w1, b1, g1, bt1, w2, b2, g2, bt2)
    gids = lax.broadcasted_iota(jnp.int32, (N, 128), 1)
    onehot = (batch_ref[...] == gids).astype(jnp.float32)
    pooled = lax.dot_general(onehot, z, (((0,), (0,)), ((), ())),
                             preferred_element_type=jnp.float32)
    out = jnp.dot(pooled, fcw_ref[...],
                  preferred_element_type=jnp.float32) + fcb_ref[...]
    o_ref[...] = out[:G, :]


_mlp_pool = pl.pallas_call(
    _mlp_pool_body,
    out_shape=jax.ShapeDtypeStruct((G, C), jnp.float32),
)


def kernel(x, edge_index, batch, W1, b1, g1, bt1, W2, b2, g2, bt2, fcW, fcb):
    src = edge_index[0].reshape(NS * NBLK, IB, K)
    dst = edge_index[1].reshape(NS * NBLK, IB, K)
    batch2d = batch.reshape(N, 1)
    hs = jnp.stack([x[:, :D2], x[:, D2:]])
    for i in range(NLAYERS):
        parts = _agg(hs, src, dst)
        wargs = (W1[i], b1[i].reshape(1, D), g1[i].reshape(1, D),
                 bt1[i].reshape(1, D),
                 W2[i], b2[i].reshape(1, D), g2[i].reshape(1, D),
                 bt2[i].reshape(1, D))
        if i + 1 < NLAYERS:
            hs = _mlp(hs, parts, *wargs)
        else:
            return _mlp_pool(hs, parts, *wargs,
                             batch2d, fcW, fcb.reshape(1, C))
